# baseline (speedup 1.0000x reference)
import functools

import jax
import jax.numpy as jnp
from jax import lax
from jax.experimental import pallas as pl
from jax.experimental.pallas import tpu as pltpu
from jax.experimental.pallas import tpu_sc as plsc

_EMBED_DIM = 128
_MAX_TIME_DIFF = 10.0
_NUM_TIME_BINS = 1024
_N = 1048576

_NC = 2
_NS = 16
_NW = _NC * _NS
_L = 16

_B_PER_W = _N // _NW
_CHUNK = 512
_N_CHUNKS = _B_PER_W // _CHUNK
_SCALE = (_NUM_TIME_BINS - 1) / _MAX_TIME_DIFF
_STEP = _MAX_TIME_DIFF / (_NUM_TIME_BINS - 1)


def _sc_body(td_hbm, table_hbm, out_hbm, td_v, idx_v, rows_v, table_spm, sem):
    sid = lax.axis_index("s")
    wid = sid * _NC + lax.axis_index("c")
    base = wid * _B_PER_W

    @pl.when(sid == 0)
    def _():
        pltpu.sync_copy(table_hbm, table_spm)

    plsc.subcore_barrier()

    def chunk_body(ci, carry):
        cbase = base + ci * _CHUNK
        pltpu.sync_copy(td_hbm.at[pl.ds(cbase, _CHUNK)], td_v)

        def vec_body(vi, c):
            off = vi * _L
            v = td_v[pl.ds(off, _L)]
            v = jnp.minimum(jnp.maximum(v, 0.0), _MAX_TIME_DIFF)
            gf = v * _SCALE
            gi = gf.astype(jnp.int32)
            gi = jnp.where(gi.astype(jnp.float32) < gf, gi + 1, gi)
            gi = jnp.clip(gi, 0, _NUM_TIME_BINS - 1)
            gif = gi.astype(jnp.float32)
            bu = gif * _STEP
            gi = jnp.where(bu < v,
                           jnp.minimum(gi + 1, _NUM_TIME_BINS - 1), gi)
            bl = (gi.astype(jnp.float32) - 1.0) * _STEP
            gi = jnp.where((gi >= 1) & (bl >= v), gi - 1, gi)
            idx_v[pl.ds(off, _L)] = gi
            return c

        lax.fori_loop(0, _CHUNK // _L, vec_body, 0)
        pltpu.async_copy(table_spm.at[idx_v], rows_v, sem).wait()
        pltpu.sync_copy(rows_v, out_hbm.at[pl.ds(cbase, _CHUNK)])
        return carry

    lax.fori_loop(0, _N_CHUNKS, chunk_body, 0)


def kernel(time_diff, time_embeddings):
    mesh = plsc.VectorSubcoreMesh(core_axis_name="c", subcore_axis_name="s")
    k = functools.partial(
        pl.kernel,
        mesh=mesh,
        out_type=jax.ShapeDtypeStruct((_N, _EMBED_DIM), jnp.float32),
        scratch_types=[
            pltpu.VMEM((_CHUNK,), jnp.float32),
            pltpu.VMEM((_CHUNK,), jnp.int32),
            pltpu.VMEM((_CHUNK, _EMBED_DIM), jnp.float32),
            pltpu.VMEM_SHARED((_NUM_TIME_BINS, _EMBED_DIM), jnp.float32),
            pltpu.SemaphoreType.DMA,
        ],
    )(_sc_body)
    return k(time_diff, time_embeddings)


# speedup vs baseline: 1.2799x; 1.2799x over previous
import functools

import jax
import jax.numpy as jnp
from jax import lax
from jax.experimental import pallas as pl
from jax.experimental.pallas import tpu as pltpu
from jax.experimental.pallas import tpu_sc as plsc

_EMBED_DIM = 128
_MAX_TIME_DIFF = 10.0
_NUM_TIME_BINS = 1024
_N = 1048576

_NC = 2
_NS = 16
_NW = _NC * _NS
_L = 16

_B_PER_W = _N // _NW
_CHUNK = 256
_N_CHUNKS = _B_PER_W // _CHUNK
_SCALE = (_NUM_TIME_BINS - 1) / _MAX_TIME_DIFF
_STEP = _MAX_TIME_DIFF / (_NUM_TIME_BINS - 1)


def _compute_idx(td_ref, idx_ref):
    def vec_body(vi, c):
        off = vi * _L
        v = td_ref[pl.ds(off, _L)]
        v = jnp.minimum(jnp.maximum(v, 0.0), _MAX_TIME_DIFF)
        gf = v * _SCALE
        gi = gf.astype(jnp.int32)
        gi = jnp.where(gi.astype(jnp.float32) < gf, gi + 1, gi)
        gi = jnp.clip(gi, 0, _NUM_TIME_BINS - 1)
        gif = gi.astype(jnp.float32)
        bu = gif * _STEP
        gi = jnp.where(bu < v, jnp.minimum(gi + 1, _NUM_TIME_BINS - 1), gi)
        bl = (gi.astype(jnp.float32) - 1.0) * _STEP
        gi = jnp.where((gi >= 1) & (bl >= v), gi - 1, gi)
        idx_ref[pl.ds(off, _L)] = gi
        return c

    lax.fori_loop(0, _CHUNK // _L, vec_body, 0)


def _sc_body(td_hbm, table_hbm, out_hbm, td_v, idx0_v, idx1_v, rows_v,
             table_spm, gsem0, gsem1, ssem0, ssem1):
    sid = lax.axis_index("s")
    wid = sid * _NC + lax.axis_index("c")
    base = wid * _B_PER_W
    gsem = (gsem0, gsem1)
    ssem = (ssem0, ssem1)
    idx_b = (idx0_v, idx1_v)

    @pl.when(sid == 0)
    def _():
        pltpu.sync_copy(table_hbm, table_spm)

    plsc.subcore_barrier()

    def prep(ci, b):
        pltpu.sync_copy(td_hbm.at[pl.ds(base + ci * _CHUNK, _CHUNK)],
                        td_v.at[b])
        _compute_idx(td_v.at[b], idx_b[b])
        pltpu.async_copy(table_spm.at[idx_b[b]], rows_v.at[b], gsem[b])

    def wait_gather(b):
        pltpu.make_async_copy(table_spm.at[idx_b[b]], rows_v.at[b],
                              gsem[b]).wait()

    def start_store(ci, b):
        pltpu.async_copy(rows_v.at[b],
                         out_hbm.at[pl.ds(base + ci * _CHUNK, _CHUNK)],
                         ssem[b])

    def wait_store(b):
        pltpu.make_async_copy(rows_v.at[b],
                              out_hbm.at[pl.ds(base, _CHUNK)],
                              ssem[b]).wait()

    prep(0, 0)

    def outer_body(oi, carry):
        for b in range(2):
            ci = oi * 2 + b
            nxt = ci + 1
            nb = 1 - b

            @pl.when(nxt < _N_CHUNKS)
            def _():
                @pl.when(nxt >= 2)
                def _():
                    wait_store(nb)

                prep(nxt, nb)

            wait_gather(b)
            start_store(ci, b)
        return carry

    lax.fori_loop(0, _N_CHUNKS // 2, outer_body, 0)
    wait_store(0)
    wait_store(1)


def kernel(time_diff, time_embeddings):
    mesh = plsc.VectorSubcoreMesh(core_axis_name="c", subcore_axis_name="s")
    k = functools.partial(
        pl.kernel,
        mesh=mesh,
        out_type=jax.ShapeDtypeStruct((_N, _EMBED_DIM), jnp.float32),
        scratch_types=[
            pltpu.VMEM((2, _CHUNK), jnp.float32),
            pltpu.VMEM((_CHUNK,), jnp.int32),
            pltpu.VMEM((_CHUNK,), jnp.int32),
            pltpu.VMEM((2, _CHUNK, _EMBED_DIM), jnp.float32),
            pltpu.VMEM_SHARED((_NUM_TIME_BINS, _EMBED_DIM), jnp.float32),
            pltpu.SemaphoreType.DMA,
            pltpu.SemaphoreType.DMA,
            pltpu.SemaphoreType.DMA,
            pltpu.SemaphoreType.DMA,
        ],
    )(_sc_body)
    return k(time_diff, time_embeddings)


# speedup vs baseline: 1.5948x; 1.2460x over previous
import functools

import jax
import jax.numpy as jnp
from jax import lax
from jax.experimental import pallas as pl
from jax.experimental.pallas import tpu as pltpu
from jax.experimental.pallas import tpu_sc as plsc

_EMBED_DIM = 128
_MAX_TIME_DIFF = 10.0
_NUM_TIME_BINS = 1024
_N = 1048576

_NC = 2
_NS = 16
_NW = _NC * _NS
_L = 16

_B_PER_W = _N // _NW
_CHUNK = 256
_N_CHUNKS = _B_PER_W // _CHUNK
_SCALE = (_NUM_TIME_BINS - 1) / _MAX_TIME_DIFF
_STEP = _MAX_TIME_DIFF / (_NUM_TIME_BINS - 1)


def _compute_idx(td_ref, idx_ref):
    def vec_body(vi, c):
        off = vi * _L
        v = td_ref[pl.ds(off, _L)]
        v = jnp.minimum(jnp.maximum(v, 0.0), _MAX_TIME_DIFF)
        gf = v * _SCALE
        gi = gf.astype(jnp.int32)
        gi = jnp.where(gi.astype(jnp.float32) < gf, gi + 1, gi)
        gi = jnp.clip(gi, 0, _NUM_TIME_BINS - 1)
        gif = gi.astype(jnp.float32)
        bu = gif * _STEP
        gi = jnp.where(bu < v, jnp.minimum(gi + 1, _NUM_TIME_BINS - 1), gi)
        bl = (gi.astype(jnp.float32) - 1.0) * _STEP
        gi = jnp.where((gi >= 1) & (bl >= v), gi - 1, gi)
        idx_ref[pl.ds(off, _L)] = gi
        return c

    lax.fori_loop(0, _CHUNK // _L, vec_body, 0)


def _sc_body(td_hbm, table_hbm, out_hbm, td_v, idx0_v, idx1_v, rows_v,
             table_spm, gsem0, gsem1, ssem0, ssem1):
    sid = lax.axis_index("s")
    wid = sid * _NC + lax.axis_index("c")
    base = wid * _B_PER_W
    gsem = (gsem0, gsem1)
    ssem = (ssem0, ssem1)
    idx_b = (idx0_v, idx1_v)

    @pl.when(sid == 0)
    def _():
        pltpu.sync_copy(table_hbm, table_spm)

    plsc.subcore_barrier()

    def prep(ci, b):
        pltpu.sync_copy(td_hbm.at[pl.ds(base + ci * _CHUNK, _CHUNK)],
                        td_v.at[b])
        _compute_idx(td_v.at[b], idx_b[b])
        pltpu.async_copy(table_spm.at[idx_b[b]], rows_v.at[b], gsem[b])

    def wait_gather(b):
        pltpu.make_async_copy(table_spm.at[idx_b[b]], rows_v.at[b],
                              gsem[b]).wait()

    def start_store(ci, b):
        pltpu.async_copy(rows_v.at[b],
                         out_hbm.at[pl.ds(base + ci * _CHUNK, _CHUNK)],
                         ssem[b])

    def wait_store(b):
        pltpu.make_async_copy(rows_v.at[b],
                              out_hbm.at[pl.ds(base, _CHUNK)],
                              ssem[b]).wait()

    prep(0, 0)

    def outer_body(oi, carry):
        for b in range(2):
            ci = oi * 2 + b
            nxt = ci + 1
            nb = 1 - b

            @pl.when(nxt < _N_CHUNKS)
            def _():
                pltpu.sync_copy(td_hbm.at[pl.ds(base + nxt * _CHUNK, _CHUNK)],
                                td_v.at[nb])
                _compute_idx(td_v.at[nb], idx_b[nb])

                @pl.when(nxt >= 2)
                def _():
                    wait_store(nb)

                pltpu.async_copy(table_spm.at[idx_b[nb]], rows_v.at[nb],
                                 gsem[nb])

            wait_gather(b)
            start_store(ci, b)
        return carry

    lax.fori_loop(0, _N_CHUNKS // 2, outer_body, 0)
    wait_store(0)
    wait_store(1)


def kernel(time_diff, time_embeddings):
    mesh = plsc.VectorSubcoreMesh(core_axis_name="c", subcore_axis_name="s")
    k = functools.partial(
        pl.kernel,
        mesh=mesh,
        out_type=jax.ShapeDtypeStruct((_N, _EMBED_DIM), jnp.float32),
        scratch_types=[
            pltpu.VMEM((2, _CHUNK), jnp.float32),
            pltpu.VMEM((_CHUNK,), jnp.int32),
            pltpu.VMEM((_CHUNK,), jnp.int32),
            pltpu.VMEM((2, _CHUNK, _EMBED_DIM), jnp.float32),
            pltpu.VMEM_SHARED((_NUM_TIME_BINS, _EMBED_DIM), jnp.float32),
            pltpu.SemaphoreType.DMA,
            pltpu.SemaphoreType.DMA,
            pltpu.SemaphoreType.DMA,
            pltpu.SemaphoreType.DMA,
        ],
    )(_sc_body)
    return k(time_diff, time_embeddings)
